# Initial kernel scaffold; baseline (speedup 1.0000x reference)
import jax
import jax.numpy as jnp
from jax.experimental import pallas as pl


def kernel(X, edge_index, edge_vals, W1, b1, W2, b2):
    raise NotImplementedError("write your pallas kernel here")


# speedup vs baseline: 3.5280x; 3.5280x over previous
import functools

import jax
import jax.numpy as jnp
from jax import lax
from jax.experimental import pallas as pl
from jax.experimental.pallas import tpu as pltpu
from jax.experimental.pallas import tpu_sc as plsc

N_NODES = 10000
D = 128
N_EDGES = 320000

NC = 2
NS = 16
NW = NC * NS
CH = 128
CHUNKS = 80
S = 40
STAGES = CHUNKS // S
EPW = CHUNKS * CH
E_PAD = NW * EPW
N_PAD = 10240
ROWS_PER_TILE = N_PAD // NS
CP = 5
CP_ROWS = ROWS_PER_TILE // CP


def _spmm_body(z_hbm, src_hbm, dst_hbm, vals_hbm, out_hbm,
               src_v, dst_v, vals_v, rows0, rows1, acc, sem0, sem1):
    cid = lax.axis_index("c")
    sid = lax.axis_index("s")
    w = cid * NS + sid

    zeros16 = jnp.zeros((16,), jnp.float32)

    def _zero_row(i, carry):
        for f in range(D // 16):
            rows0[i, pl.ds(f * 16, 16)] = zeros16
        return carry

    lax.fori_loop(0, CH, _zero_row, 0)
    base_row = sid * ROWS_PER_TILE
    for k in range(CP):
        pltpu.sync_copy(rows0.at[pl.ds(0, CP_ROWS)],
                        acc.at[pl.ds(base_row + k * CP_ROWS, CP_ROWS)])
    plsc.subcore_barrier()

    bufs = (rows0, rows1)
    sems = (sem0, sem1)

    def _scale_chunk(rows, j):
        def _group(g, carry):
            vals16 = vals_v[j, pl.ds(g * 16, 16)]
            for l in range(16):
                vv = jnp.full((16,), vals16[l], jnp.float32)
                e = g * 16 + l
                for f in range(D // 16):
                    sl = pl.ds(f * 16, 16)
                    rows[e, sl] = rows[e, sl] * vv
            return carry
        lax.fori_loop(0, CH // 16, _group, 0)

    def _stage(s, carry):
        pltpu.sync_copy(src_hbm.at[w, pl.ds(s * S, S)], src_v)
        pltpu.sync_copy(dst_hbm.at[w, pl.ds(s * S, S)], dst_v)
        pltpu.sync_copy(vals_hbm.at[w, pl.ds(s * S, S)], vals_v)

        pltpu.async_copy(z_hbm.at[src_v.at[0]], rows0, sem0)
        pltpu.async_copy(z_hbm.at[src_v.at[1]], rows1, sem1)

        def _outer(t, carry2):
            for b in range(2):
                j = 2 * t + b
                rows = bufs[b]
                sem = sems[b]
                pltpu.make_async_copy(z_hbm.at[src_v.at[j]], rows, sem).wait()
                _scale_chunk(rows, j)
                pltpu.sync_copy(rows, acc.at[dst_v.at[j]], add=True)
                @pl.when(j + 2 < S)
                def _():
                    pltpu.async_copy(z_hbm.at[src_v.at[j + 2]], rows, sem)
            return carry2

        lax.fori_loop(0, S // 2, _outer, 0)
        return carry

    lax.fori_loop(0, STAGES, _stage, 0)
    plsc.subcore_barrier()

    for k in range(CP):
        r0 = base_row + k * CP_ROWS
        pltpu.sync_copy(acc.at[pl.ds(r0, CP_ROWS)], rows0.at[pl.ds(0, CP_ROWS)])
        pltpu.sync_copy(rows0.at[pl.ds(0, CP_ROWS)], out_hbm.at[cid, pl.ds(r0, CP_ROWS)])


_spmm = functools.partial(
    pl.kernel,
    out_type=jax.ShapeDtypeStruct((NC, N_PAD, D), jnp.float32),
    mesh=plsc.VectorSubcoreMesh(core_axis_name="c", subcore_axis_name="s"),
    scratch_types=[
        pltpu.VMEM((S, CH), jnp.int32),
        pltpu.VMEM((S, CH), jnp.int32),
        pltpu.VMEM((S, CH), jnp.float32),
        pltpu.VMEM((CH, D), jnp.float32),
        pltpu.VMEM((CH, D), jnp.float32),
        pltpu.VMEM_SHARED((N_PAD, D), jnp.float32),
        pltpu.SemaphoreType.DMA,
        pltpu.SemaphoreType.DMA,
    ],
)(_spmm_body)


_RB = 1000
_RG = N_NODES // _RB


def _linear_body(x_ref, w_ref, b_ref, o_ref):
    o_ref[...] = (
        jnp.dot(x_ref[...], w_ref[...], preferred_element_type=jnp.float32)
        + b_ref[...]
    )


def _linear(x, wt, b2d):
    return pl.pallas_call(
        _linear_body,
        grid=(_RG,),
        in_specs=[
            pl.BlockSpec((_RB, D), lambda i: (i, 0)),
            pl.BlockSpec((D, D), lambda i: (0, 0)),
            pl.BlockSpec((1, D), lambda i: (0, 0)),
        ],
        out_specs=pl.BlockSpec((_RB, D), lambda i: (i, 0)),
        out_shape=jax.ShapeDtypeStruct((N_NODES, D), jnp.float32),
    )(x, wt, b2d)


def _fuse_body(p_ref, w_ref, b_ref, o_ref):
    x = jnp.maximum(p_ref[0] + p_ref[1], 0.0)
    o_ref[...] = (
        jnp.dot(x, w_ref[...], preferred_element_type=jnp.float32) + b_ref[...]
    )


def _relu_sum_linear(p, wt, b2d):
    return pl.pallas_call(
        _fuse_body,
        grid=(_RG,),
        in_specs=[
            pl.BlockSpec((NC, _RB, D), lambda i: (0, i, 0)),
            pl.BlockSpec((D, D), lambda i: (0, 0)),
            pl.BlockSpec((1, D), lambda i: (0, 0)),
        ],
        out_specs=pl.BlockSpec((_RB, D), lambda i: (i, 0)),
        out_shape=jax.ShapeDtypeStruct((N_NODES, D), jnp.float32),
    )(p, wt, b2d)


def _sum_body(p_ref, o_ref):
    o_ref[...] = p_ref[0] + p_ref[1]


def _sum_partials(p):
    return pl.pallas_call(
        _sum_body,
        grid=(_RG,),
        in_specs=[pl.BlockSpec((NC, _RB, D), lambda i: (0, i, 0))],
        out_specs=pl.BlockSpec((_RB, D), lambda i: (i, 0)),
        out_shape=jax.ShapeDtypeStruct((N_NODES, D), jnp.float32),
    )(p)


def kernel(X, edge_index, edge_vals, W1, b1, W2, b2):
    pad = E_PAD - N_EDGES
    src = jnp.concatenate([edge_index[1], jnp.zeros((pad,), jnp.int32)])
    dst = jnp.concatenate([edge_index[0], jnp.zeros((pad,), jnp.int32)])
    vals = jnp.concatenate([edge_vals, jnp.zeros((pad,), jnp.float32)])
    src3 = src.reshape(NW, CHUNKS, CH)
    dst3 = dst.reshape(NW, CHUNKS, CH)
    vals3 = vals.reshape(NW, CHUNKS, CH)

    w1t = W1.T
    w2t = W2.T
    b1_2d = b1.reshape(1, D)
    b2_2d = b2.reshape(1, D)

    z1 = _linear(X, w1t, b1_2d)
    p1 = _spmm(z1, src3, dst3, vals3)
    z2 = _relu_sum_linear(p1, w2t, b2_2d)
    p2 = _spmm(z2, src3, dst3, vals3)
    return _sum_partials(p2)


# speedup vs baseline: 10.9917x; 3.1156x over previous
import functools

import jax
import jax.numpy as jnp
from jax import lax
from jax.experimental import pallas as pl
from jax.experimental.pallas import tpu as pltpu
from jax.experimental.pallas import tpu_sc as plsc

N_NODES = 10000
D = 128
N_EDGES = 320000

NC = 2
NS = 16
NW = NC * NS
CH = 128
CHUNKS = 80
S = 40
STAGES = CHUNKS // S
EPW = CHUNKS * CH
E_PAD = NW * EPW
N_PAD = 10240
ROWS_PER_TILE = N_PAD // NS
CP = 5
CP_ROWS = ROWS_PER_TILE // CP


def _spmm_body(z_hbm, src_hbm, dst_hbm, vals_hbm, out_hbm,
               src_v, dst_v, vals_v, rows0, rows1, acc, sem0, sem1):
    cid = lax.axis_index("c")
    sid = lax.axis_index("s")
    w = cid * NS + sid

    zeros16 = jnp.zeros((16,), jnp.float32)

    def _zero_row(i, carry):
        for f in range(D // 16):
            rows0[i, pl.ds(f * 16, 16)] = zeros16
        return carry

    lax.fori_loop(0, CH, _zero_row, 0)
    base_row = sid * ROWS_PER_TILE
    for k in range(CP):
        pltpu.sync_copy(rows0.at[pl.ds(0, CP_ROWS)],
                        acc.at[pl.ds(base_row + k * CP_ROWS, CP_ROWS)])
    plsc.subcore_barrier()

    bufs = (rows0, rows1)
    sems = (sem0, sem1)

    def _scale_chunk(rows, j):
        def _group(g, carry):
            vals16 = vals_v[j, pl.ds(g * 16, 16)]
            for l in range(16):
                vv = jnp.full((16,), vals16[l], jnp.float32)
                e = g * 16 + l
                for f in range(D // 16):
                    sl = pl.ds(f * 16, 16)
                    rows[e, sl] = rows[e, sl] * vv
            return carry
        lax.fori_loop(0, CH // 16, _group, 0)

    def _stage(s, carry):
        pltpu.sync_copy(src_hbm.at[w, pl.ds(s * S, S)], src_v)
        pltpu.sync_copy(dst_hbm.at[w, pl.ds(s * S, S)], dst_v)
        pltpu.sync_copy(vals_hbm.at[w, pl.ds(s * S, S)], vals_v)

        pltpu.async_copy(z_hbm.at[src_v.at[0]], rows0, sem0)
        pltpu.async_copy(z_hbm.at[src_v.at[1]], rows1, sem1)

        def _outer(t, carry2):
            for b in range(2):
                j = 2 * t + b
                rows = bufs[b]
                sem = sems[b]
                pltpu.make_async_copy(z_hbm.at[src_v.at[j]], rows, sem).wait()
                _scale_chunk(rows, j)
                pltpu.sync_copy(rows, acc.at[dst_v.at[j]], add=True)
                @pl.when(j + 2 < S)
                def _():
                    pltpu.async_copy(z_hbm.at[src_v.at[j + 2]], rows, sem)
            return carry2

        lax.fori_loop(0, S // 2, _outer, 0)
        return carry

    lax.fori_loop(0, STAGES, _stage, 0)
    plsc.subcore_barrier()

    for k in range(CP):
        r0 = base_row + k * CP_ROWS
        pltpu.sync_copy(acc.at[pl.ds(r0, CP_ROWS)], rows0.at[pl.ds(0, CP_ROWS)])
        pltpu.sync_copy(rows0.at[pl.ds(0, CP_ROWS)], out_hbm.at[cid, pl.ds(r0, CP_ROWS)])


_spmm = functools.partial(
    pl.kernel,
    out_type=jax.ShapeDtypeStruct((NC, N_PAD, D), jnp.float32),
    mesh=plsc.VectorSubcoreMesh(core_axis_name="c", subcore_axis_name="s"),
    scratch_types=[
        pltpu.VMEM((S, CH), jnp.int32),
        pltpu.VMEM((S, CH), jnp.int32),
        pltpu.VMEM((S, CH), jnp.float32),
        pltpu.VMEM((CH, D), jnp.float32),
        pltpu.VMEM((CH, D), jnp.float32),
        pltpu.VMEM_SHARED((N_PAD, D), jnp.float32),
        pltpu.SemaphoreType.DMA,
        pltpu.SemaphoreType.DMA,
    ],
)(_spmm_body)


_RB = 1000
_RG = N_NODES // _RB


def _linear_body(x_ref, w_ref, b_ref, o_ref):
    o_ref[...] = (
        jnp.dot(x_ref[...], w_ref[...], preferred_element_type=jnp.float32)
        + b_ref[...]
    )


def _linear(x, wt, b2d):
    return pl.pallas_call(
        _linear_body,
        grid=(_RG,),
        in_specs=[
            pl.BlockSpec((_RB, D), lambda i: (i, 0)),
            pl.BlockSpec((D, D), lambda i: (0, 0)),
            pl.BlockSpec((1, D), lambda i: (0, 0)),
        ],
        out_specs=pl.BlockSpec((_RB, D), lambda i: (i, 0)),
        out_shape=jax.ShapeDtypeStruct((N_NODES, D), jnp.float32),
    )(x, wt, b2d)


def _fuse_body(p_ref, w_ref, b_ref, o_ref):
    x = jnp.maximum(p_ref[0] + p_ref[1], 0.0)
    o_ref[...] = (
        jnp.dot(x, w_ref[...], preferred_element_type=jnp.float32) + b_ref[...]
    )


def _relu_sum_linear(p, wt, b2d):
    return pl.pallas_call(
        _fuse_body,
        grid=(_RG,),
        in_specs=[
            pl.BlockSpec((NC, _RB, D), lambda i: (0, i, 0)),
            pl.BlockSpec((D, D), lambda i: (0, 0)),
            pl.BlockSpec((1, D), lambda i: (0, 0)),
        ],
        out_specs=pl.BlockSpec((_RB, D), lambda i: (i, 0)),
        out_shape=jax.ShapeDtypeStruct((N_NODES, D), jnp.float32),
    )(p, wt, b2d)


def _sum_body(p_ref, o_ref):
    o_ref[...] = p_ref[0] + p_ref[1]


def _sum_partials(p):
    return pl.pallas_call(
        _sum_body,
        grid=(_RG,),
        in_specs=[pl.BlockSpec((NC, _RB, D), lambda i: (0, i, 0))],
        out_specs=pl.BlockSpec((_RB, D), lambda i: (i, 0)),
        out_shape=jax.ShapeDtypeStruct((N_NODES, D), jnp.float32),
    )(p)


def kernel(X, edge_index, edge_vals, W1, b1, W2, b2):
    pad = E_PAD - N_EDGES
    pad_ids = jnp.arange(pad, dtype=jnp.int32)
    src = jnp.concatenate([edge_index[1], pad_ids % N_NODES])
    dst = jnp.concatenate([edge_index[0], N_NODES + pad_ids % (N_PAD - N_NODES)])
    vals = jnp.concatenate([edge_vals, jnp.zeros((pad,), jnp.float32)])
    src3 = src.reshape(NW, CHUNKS, CH)
    dst3 = dst.reshape(NW, CHUNKS, CH)
    vals3 = vals.reshape(NW, CHUNKS, CH)

    w1t = W1.T
    w2t = W2.T
    b1_2d = b1.reshape(1, D)
    b2_2d = b2.reshape(1, D)

    z1 = _linear(X, w1t, b1_2d)
    p1 = _spmm(z1, src3, dst3, vals3)
    z2 = _relu_sum_linear(p1, w2t, b2_2d)
    p2 = _spmm(z2, src3, dst3, vals3)
    return _sum_partials(p2)


# speedup vs baseline: 11.8598x; 1.0790x over previous
import functools

import jax
import jax.numpy as jnp
from jax import lax
from jax.experimental import pallas as pl
from jax.experimental.pallas import tpu as pltpu
from jax.experimental.pallas import tpu_sc as plsc

N_NODES = 10000
D = 128
N_EDGES = 320000

NC = 2
NS = 16
NW = NC * NS
CH = 112
CHUNKS = 96
S = 8
STAGES = CHUNKS // S
EPW = CHUNKS * CH
E_PAD = NW * EPW
N_PAD = 10112
ROWS_PER_TILE = N_PAD // NS
CP_SIZES = (112, 112, 112, 112, 112, 72)


def _spmm_body(z_hbm, src_hbm, dst_hbm, vals_hbm, out_hbm,
               src_t, dst_t, vals_t, rows0, rows1, rows2,
               acc, g0, g1, g2, s0, s1, s2, tsem, osem):
    cid = lax.axis_index("c")
    sid = lax.axis_index("s")
    w = cid * NS + sid

    bufs = (rows0, rows1, rows2)
    gsems = (g0, g1, g2)
    ssems = (s0, s1, s2)

    zeros16 = jnp.zeros((16,), jnp.float32)

    def _zero_row(i, carry):
        for f in range(D // 16):
            rows0[i, pl.ds(f * 16, 16)] = zeros16
        return carry

    lax.fori_loop(0, CH, _zero_row, 0)
    base_row = sid * ROWS_PER_TILE
    off = 0
    for sz in CP_SIZES:
        pltpu.async_copy(rows0.at[pl.ds(0, sz)],
                         acc.at[pl.ds(base_row + off, sz)], osem)
        off += sz
    off = 0
    for sz in CP_SIZES:
        pltpu.make_async_copy(rows0.at[pl.ds(0, sz)],
                              acc.at[pl.ds(base_row + off, sz)], osem).wait()
        off += sz
    plsc.subcore_barrier()

    pltpu.sync_copy(src_hbm.at[w, pl.ds(0, S)], src_t.at[pl.ds(0, S)])
    pltpu.sync_copy(dst_hbm.at[w, pl.ds(0, S)], dst_t.at[pl.ds(0, S)])
    pltpu.sync_copy(vals_hbm.at[w, pl.ds(0, S)], vals_t.at[pl.ds(0, S)])
    pltpu.async_copy(src_hbm.at[w, pl.ds(S, S)], src_t.at[pl.ds(S, S)], tsem)
    pltpu.async_copy(dst_hbm.at[w, pl.ds(S, S)], dst_t.at[pl.ds(S, S)], tsem)
    pltpu.async_copy(vals_hbm.at[w, pl.ds(S, S)], vals_t.at[pl.ds(S, S)], tsem)

    pltpu.async_copy(z_hbm.at[src_t.at[0]], rows0, g0)
    pltpu.async_copy(z_hbm.at[src_t.at[1]], rows1, g1)

    def _scale_chunk(rows, r):
        def _group(g, carry):
            vals16 = vals_t[r, pl.ds(g * 16, 16)]
            for l in range(16):
                vv = jnp.full((16,), vals16[l], jnp.float32)
                e = g * 16 + l
                for f in range(D // 16):
                    sl = pl.ds(f * 16, 16)
                    rows[e, sl] = rows[e, sl] * vv
            return carry
        lax.fori_loop(0, CH // 16, _group, 0)

    def _outer(t, carry):
        for b in range(3):
            j = 3 * t + b
            rows = bufs[b]
            r = j % (2 * S)
            pltpu.make_async_copy(z_hbm.at[src_t.at[r]], rows, gsems[b]).wait()
            _scale_chunk(rows, r)
            pltpu.async_copy(rows, acc.at[dst_t.at[r]], ssems[b], add=True)

            b2 = (b + 2) % 3
            rows_n = bufs[b2]

            @pl.when(j + 2 < CHUNKS)
            def _():
                @pl.when(j > 0)
                def _():
                    pltpu.make_async_copy(
                        rows_n, acc.at[dst_t.at[(j - 1) % (2 * S)]],
                        ssems[b2]).wait()

                @pl.when((j + 2) % S == 0)
                def _():
                    sc2 = (j + 2) // S
                    toff = (sc2 % 2) * S
                    hoff = sc2 * S
                    pltpu.make_async_copy(
                        src_hbm.at[w, pl.ds(hoff, S)],
                        src_t.at[pl.ds(toff, S)], tsem).wait()
                    pltpu.make_async_copy(
                        dst_hbm.at[w, pl.ds(hoff, S)],
                        dst_t.at[pl.ds(toff, S)], tsem).wait()
                    pltpu.make_async_copy(
                        vals_hbm.at[w, pl.ds(hoff, S)],
                        vals_t.at[pl.ds(toff, S)], tsem).wait()

                pltpu.async_copy(z_hbm.at[src_t.at[(j + 2) % (2 * S)]],
                                 rows_n, gsems[b2])

            @pl.when((j % S == 0) & (j > 0) & (j + S < CHUNKS))
            def _():
                sc1 = j // S + 1
                toff = (sc1 % 2) * S
                hoff = sc1 * S
                pltpu.async_copy(src_hbm.at[w, pl.ds(hoff, S)],
                                 src_t.at[pl.ds(toff, S)], tsem)
                pltpu.async_copy(dst_hbm.at[w, pl.ds(hoff, S)],
                                 dst_t.at[pl.ds(toff, S)], tsem)
                pltpu.async_copy(vals_hbm.at[w, pl.ds(hoff, S)],
                                 vals_t.at[pl.ds(toff, S)], tsem)
        return carry

    lax.fori_loop(0, CHUNKS // 3, _outer, 0)

    for jj in (CHUNKS - 3, CHUNKS - 2, CHUNKS - 1):
        pltpu.make_async_copy(bufs[jj % 3],
                              acc.at[dst_t.at[jj % (2 * S)]],
                              ssems[jj % 3]).wait()
    plsc.subcore_barrier()

    offs = []
    off = 0
    for sz in CP_SIZES:
        offs.append(off)
        off += sz
    for k in range(3):
        pltpu.async_copy(acc.at[pl.ds(base_row + offs[k], CP_SIZES[k])],
                         bufs[k].at[pl.ds(0, CP_SIZES[k])], gsems[k])
    for k, sz in enumerate(CP_SIZES):
        b = k % 3
        pltpu.make_async_copy(acc.at[pl.ds(base_row + offs[k], sz)],
                              bufs[b].at[pl.ds(0, sz)], gsems[b]).wait()
        pltpu.async_copy(bufs[b].at[pl.ds(0, sz)],
                         out_hbm.at[cid, pl.ds(base_row + offs[k], sz)],
                         ssems[b])
        if k + 3 < len(CP_SIZES):
            pltpu.make_async_copy(bufs[b].at[pl.ds(0, sz)],
                                  out_hbm.at[cid, pl.ds(base_row + offs[k], sz)],
                                  ssems[b]).wait()
            pltpu.async_copy(acc.at[pl.ds(base_row + offs[k + 3], CP_SIZES[k + 3])],
                             bufs[b].at[pl.ds(0, CP_SIZES[k + 3])], gsems[b])
    for k in range(3, len(CP_SIZES)):
        b = k % 3
        sz = CP_SIZES[k]
        pltpu.make_async_copy(bufs[b].at[pl.ds(0, sz)],
                              out_hbm.at[cid, pl.ds(base_row + offs[k], sz)],
                              ssems[b]).wait()


_spmm = functools.partial(
    pl.kernel,
    out_type=jax.ShapeDtypeStruct((NC, N_PAD, D), jnp.float32),
    mesh=plsc.VectorSubcoreMesh(core_axis_name="c", subcore_axis_name="s"),
    scratch_types=[
        pltpu.VMEM((2 * S, CH), jnp.int32),
        pltpu.VMEM((2 * S, CH), jnp.int32),
        pltpu.VMEM((2 * S, CH), jnp.float32),
        pltpu.VMEM((CH, D), jnp.float32),
        pltpu.VMEM((CH, D), jnp.float32),
        pltpu.VMEM((CH, D), jnp.float32),
        pltpu.VMEM_SHARED((N_PAD, D), jnp.float32),
        pltpu.SemaphoreType.DMA,
        pltpu.SemaphoreType.DMA,
        pltpu.SemaphoreType.DMA,
        pltpu.SemaphoreType.DMA,
        pltpu.SemaphoreType.DMA,
        pltpu.SemaphoreType.DMA,
        pltpu.SemaphoreType.DMA,
        pltpu.SemaphoreType.DMA,
    ],
)(_spmm_body)


_RB = 1000
_RG = N_NODES // _RB


def _linear_body(x_ref, w_ref, b_ref, o_ref):
    o_ref[...] = (
        jnp.dot(x_ref[...], w_ref[...], preferred_element_type=jnp.float32)
        + b_ref[...]
    )


def _linear(x, wt, b2d):
    return pl.pallas_call(
        _linear_body,
        grid=(_RG,),
        in_specs=[
            pl.BlockSpec((_RB, D), lambda i: (i, 0)),
            pl.BlockSpec((D, D), lambda i: (0, 0)),
            pl.BlockSpec((1, D), lambda i: (0, 0)),
        ],
        out_specs=pl.BlockSpec((_RB, D), lambda i: (i, 0)),
        out_shape=jax.ShapeDtypeStruct((N_NODES, D), jnp.float32),
    )(x, wt, b2d)


def _fuse_body(p_ref, w_ref, b_ref, o_ref):
    x = jnp.maximum(p_ref[0] + p_ref[1], 0.0)
    o_ref[...] = (
        jnp.dot(x, w_ref[...], preferred_element_type=jnp.float32) + b_ref[...]
    )


def _relu_sum_linear(p, wt, b2d):
    return pl.pallas_call(
        _fuse_body,
        grid=(_RG,),
        in_specs=[
            pl.BlockSpec((NC, _RB, D), lambda i: (0, i, 0)),
            pl.BlockSpec((D, D), lambda i: (0, 0)),
            pl.BlockSpec((1, D), lambda i: (0, 0)),
        ],
        out_specs=pl.BlockSpec((_RB, D), lambda i: (i, 0)),
        out_shape=jax.ShapeDtypeStruct((N_NODES, D), jnp.float32),
    )(p, wt, b2d)


def _sum_body(p_ref, o_ref):
    o_ref[...] = p_ref[0] + p_ref[1]


def _sum_partials(p):
    return pl.pallas_call(
        _sum_body,
        grid=(_RG,),
        in_specs=[pl.BlockSpec((NC, _RB, D), lambda i: (0, i, 0))],
        out_specs=pl.BlockSpec((_RB, D), lambda i: (i, 0)),
        out_shape=jax.ShapeDtypeStruct((N_NODES, D), jnp.float32),
    )(p)


def kernel(X, edge_index, edge_vals, W1, b1, W2, b2):
    pad = E_PAD - N_EDGES
    pad_ids = jnp.arange(pad, dtype=jnp.int32)
    src = jnp.concatenate([edge_index[1], pad_ids % N_NODES])
    dst = jnp.concatenate([edge_index[0], N_NODES + pad_ids % (N_PAD - N_NODES)])
    vals = jnp.concatenate([edge_vals, jnp.zeros((pad,), jnp.float32)])
    src3 = src.reshape(NW, CHUNKS, CH)
    dst3 = dst.reshape(NW, CHUNKS, CH)
    vals3 = vals.reshape(NW, CHUNKS, CH)

    w1t = W1.T
    w2t = W2.T
    b1_2d = b1.reshape(1, D)
    b2_2d = b2.reshape(1, D)

    z1 = _linear(X, w1t, b1_2d)
    p1 = _spmm(z1, src3, dst3, vals3)
    z2 = _relu_sum_linear(p1, w2t, b2_2d)
    p2 = _spmm(z2, src3, dst3, vals3)
    return _sum_partials(p2)


# speedup vs baseline: 12.3373x; 1.0403x over previous
import functools

import jax
import jax.numpy as jnp
from jax import lax
from jax.experimental import pallas as pl
from jax.experimental.pallas import tpu as pltpu
from jax.experimental.pallas import tpu_sc as plsc

N_NODES = 10000
D = 128
N_EDGES = 320000

NC = 2
NS = 16
NW = NC * NS
CH = 112
CHUNKS = 96
S = 8
STAGES = CHUNKS // S
EPW = CHUNKS * CH
E_PAD = NW * EPW
N_PAD = 10112
ROWS_PER_TILE = N_PAD // NS
CP_SIZES = (112, 112, 112, 112, 112, 72)


def _spmm_body(z_hbm, src_hbm, dst_hbm, vals_hbm, out_hbm,
               src_t, dst_t, vals_t, rows0, rows1, rows2,
               acc, g0, g1, g2, s0, s1, s2, tsem, osem):
    cid = lax.axis_index("c")
    sid = lax.axis_index("s")
    w = cid * NS + sid

    bufs = (rows0, rows1, rows2)
    gsems = (g0, g1, g2)
    ssems = (s0, s1, s2)

    zeros16 = jnp.zeros((16,), jnp.float32)

    def _zero_row(i, carry):
        for f in range(D // 16):
            rows0[i, pl.ds(f * 16, 16)] = zeros16
        return carry

    lax.fori_loop(0, CH, _zero_row, 0)
    base_row = sid * ROWS_PER_TILE
    off = 0
    for sz in CP_SIZES:
        pltpu.async_copy(rows0.at[pl.ds(0, sz)],
                         acc.at[pl.ds(base_row + off, sz)], osem)
        off += sz
    pltpu.sync_copy(src_hbm.at[w, pl.ds(0, S)], src_t.at[pl.ds(0, S)])
    pltpu.sync_copy(dst_hbm.at[w, pl.ds(0, S)], dst_t.at[pl.ds(0, S)])
    pltpu.sync_copy(vals_hbm.at[w, pl.ds(0, S)], vals_t.at[pl.ds(0, S)])
    pltpu.async_copy(src_hbm.at[w, pl.ds(S, S)], src_t.at[pl.ds(S, S)], tsem)
    pltpu.async_copy(dst_hbm.at[w, pl.ds(S, S)], dst_t.at[pl.ds(S, S)], tsem)
    pltpu.async_copy(vals_hbm.at[w, pl.ds(S, S)], vals_t.at[pl.ds(S, S)], tsem)

    off = 0
    for sz in CP_SIZES:
        pltpu.make_async_copy(rows0.at[pl.ds(0, sz)],
                              acc.at[pl.ds(base_row + off, sz)], osem).wait()
        off += sz

    pltpu.async_copy(z_hbm.at[src_t.at[0]], rows0, g0)
    pltpu.async_copy(z_hbm.at[src_t.at[1]], rows1, g1)
    plsc.subcore_barrier()

    def _scale_chunk(rows, r):
        def _group(g, carry):
            vals16 = vals_t[r, pl.ds(g * 16, 16)]
            for l in range(16):
                vv = jnp.full((16,), vals16[l], jnp.float32)
                e = g * 16 + l
                for f in range(D // 16):
                    sl = pl.ds(f * 16, 16)
                    rows[e, sl] = rows[e, sl] * vv
            return carry
        lax.fori_loop(0, CH // 16, _group, 0)

    def _outer(t, carry):
        for b in range(3):
            j = 3 * t + b
            rows = bufs[b]
            r = j % (2 * S)
            pltpu.make_async_copy(z_hbm.at[src_t.at[r]], rows, gsems[b]).wait()
            _scale_chunk(rows, r)
            pltpu.async_copy(rows, acc.at[dst_t.at[r]], ssems[b], add=True)

            b2 = (b + 2) % 3
            rows_n = bufs[b2]

            @pl.when(j + 2 < CHUNKS)
            def _():
                @pl.when(j > 0)
                def _():
                    pltpu.make_async_copy(
                        rows_n, acc.at[dst_t.at[(j - 1) % (2 * S)]],
                        ssems[b2]).wait()

                @pl.when((j + 2) % S == 0)
                def _():
                    sc2 = (j + 2) // S
                    toff = (sc2 % 2) * S
                    hoff = sc2 * S
                    pltpu.make_async_copy(
                        src_hbm.at[w, pl.ds(hoff, S)],
                        src_t.at[pl.ds(toff, S)], tsem).wait()
                    pltpu.make_async_copy(
                        dst_hbm.at[w, pl.ds(hoff, S)],
                        dst_t.at[pl.ds(toff, S)], tsem).wait()
                    pltpu.make_async_copy(
                        vals_hbm.at[w, pl.ds(hoff, S)],
                        vals_t.at[pl.ds(toff, S)], tsem).wait()

                pltpu.async_copy(z_hbm.at[src_t.at[(j + 2) % (2 * S)]],
                                 rows_n, gsems[b2])

            @pl.when((j % S == 0) & (j > 0) & (j + S < CHUNKS))
            def _():
                sc1 = j // S + 1
                toff = (sc1 % 2) * S
                hoff = sc1 * S
                pltpu.async_copy(src_hbm.at[w, pl.ds(hoff, S)],
                                 src_t.at[pl.ds(toff, S)], tsem)
                pltpu.async_copy(dst_hbm.at[w, pl.ds(hoff, S)],
                                 dst_t.at[pl.ds(toff, S)], tsem)
                pltpu.async_copy(vals_hbm.at[w, pl.ds(hoff, S)],
                                 vals_t.at[pl.ds(toff, S)], tsem)
        return carry

    lax.fori_loop(0, CHUNKS // 3, _outer, 0)

    for jj in (CHUNKS - 3, CHUNKS - 2, CHUNKS - 1):
        pltpu.make_async_copy(bufs[jj % 3],
                              acc.at[dst_t.at[jj % (2 * S)]],
                              ssems[jj % 3]).wait()
    plsc.subcore_barrier()

    offs = []
    off = 0
    for sz in CP_SIZES:
        offs.append(off)
        off += sz
    for k in range(3):
        pltpu.async_copy(acc.at[pl.ds(base_row + offs[k], CP_SIZES[k])],
                         bufs[k].at[pl.ds(0, CP_SIZES[k])], gsems[k])
    for k, sz in enumerate(CP_SIZES):
        b = k % 3
        pltpu.make_async_copy(acc.at[pl.ds(base_row + offs[k], sz)],
                              bufs[b].at[pl.ds(0, sz)], gsems[b]).wait()
        pltpu.async_copy(bufs[b].at[pl.ds(0, sz)],
                         out_hbm.at[cid, pl.ds(base_row + offs[k], sz)],
                         ssems[b])
        if k + 3 < len(CP_SIZES):
            pltpu.make_async_copy(bufs[b].at[pl.ds(0, sz)],
                                  out_hbm.at[cid, pl.ds(base_row + offs[k], sz)],
                                  ssems[b]).wait()
            pltpu.async_copy(acc.at[pl.ds(base_row + offs[k + 3], CP_SIZES[k + 3])],
                             bufs[b].at[pl.ds(0, CP_SIZES[k + 3])], gsems[b])
    for k in range(3, len(CP_SIZES)):
        b = k % 3
        sz = CP_SIZES[k]
        pltpu.make_async_copy(bufs[b].at[pl.ds(0, sz)],
                              out_hbm.at[cid, pl.ds(base_row + offs[k], sz)],
                              ssems[b]).wait()


_spmm = functools.partial(
    pl.kernel,
    out_type=jax.ShapeDtypeStruct((NC, N_PAD, D), jnp.float32),
    mesh=plsc.VectorSubcoreMesh(core_axis_name="c", subcore_axis_name="s"),
    scratch_types=[
        pltpu.VMEM((2 * S, CH), jnp.int32),
        pltpu.VMEM((2 * S, CH), jnp.int32),
        pltpu.VMEM((2 * S, CH), jnp.float32),
        pltpu.VMEM((CH, D), jnp.float32),
        pltpu.VMEM((CH, D), jnp.float32),
        pltpu.VMEM((CH, D), jnp.float32),
        pltpu.VMEM_SHARED((N_PAD, D), jnp.float32),
        pltpu.SemaphoreType.DMA,
        pltpu.SemaphoreType.DMA,
        pltpu.SemaphoreType.DMA,
        pltpu.SemaphoreType.DMA,
        pltpu.SemaphoreType.DMA,
        pltpu.SemaphoreType.DMA,
        pltpu.SemaphoreType.DMA,
        pltpu.SemaphoreType.DMA,
    ],
)(_spmm_body)


_RB = 2000
_RG = N_NODES // _RB


def _linear_body(x_ref, w_ref, b_ref, o_ref):
    o_ref[...] = (
        jnp.dot(x_ref[...], w_ref[...], preferred_element_type=jnp.float32)
        + b_ref[...]
    )


def _linear(x, wt, b2d):
    return pl.pallas_call(
        _linear_body,
        grid=(_RG,),
        in_specs=[
            pl.BlockSpec((_RB, D), lambda i: (i, 0)),
            pl.BlockSpec((D, D), lambda i: (0, 0)),
            pl.BlockSpec((1, D), lambda i: (0, 0)),
        ],
        out_specs=pl.BlockSpec((_RB, D), lambda i: (i, 0)),
        out_shape=jax.ShapeDtypeStruct((N_NODES, D), jnp.float32),
    )(x, wt, b2d)


def _fuse_body(p_ref, w_ref, b_ref, o_ref):
    x = jnp.maximum(p_ref[0] + p_ref[1], 0.0)
    o_ref[...] = (
        jnp.dot(x, w_ref[...], preferred_element_type=jnp.float32) + b_ref[...]
    )


def _relu_sum_linear(p, wt, b2d):
    return pl.pallas_call(
        _fuse_body,
        grid=(_RG,),
        in_specs=[
            pl.BlockSpec((NC, _RB, D), lambda i: (0, i, 0)),
            pl.BlockSpec((D, D), lambda i: (0, 0)),
            pl.BlockSpec((1, D), lambda i: (0, 0)),
        ],
        out_specs=pl.BlockSpec((_RB, D), lambda i: (i, 0)),
        out_shape=jax.ShapeDtypeStruct((N_NODES, D), jnp.float32),
    )(p, wt, b2d)


def _sum_body(p_ref, o_ref):
    o_ref[...] = p_ref[0] + p_ref[1]


def _sum_partials(p):
    return pl.pallas_call(
        _sum_body,
        grid=(_RG,),
        in_specs=[pl.BlockSpec((NC, _RB, D), lambda i: (0, i, 0))],
        out_specs=pl.BlockSpec((_RB, D), lambda i: (i, 0)),
        out_shape=jax.ShapeDtypeStruct((N_NODES, D), jnp.float32),
    )(p)


def kernel(X, edge_index, edge_vals, W1, b1, W2, b2):
    pad = E_PAD - N_EDGES
    pad_ids = jnp.arange(pad, dtype=jnp.int32)
    src = jnp.concatenate([edge_index[1], pad_ids % N_NODES])
    dst = jnp.concatenate([edge_index[0], N_NODES + pad_ids % (N_PAD - N_NODES)])
    vals = jnp.concatenate([edge_vals, jnp.zeros((pad,), jnp.float32)])
    src3 = src.reshape(NW, CHUNKS, CH)
    dst3 = dst.reshape(NW, CHUNKS, CH)
    vals3 = vals.reshape(NW, CHUNKS, CH)

    w1t = W1.T
    w2t = W2.T
    b1_2d = b1.reshape(1, D)
    b2_2d = b2.reshape(1, D)

    z1 = _linear(X, w1t, b1_2d)
    p1 = _spmm(z1, src3, dst3, vals3)
    z2 = _relu_sum_linear(p1, w2t, b2_2d)
    p2 = _spmm(z2, src3, dst3, vals3)
    return _sum_partials(p2)
